# baseline (speedup 1.0000x reference)
import jax
import jax.numpy as jnp
from jax.experimental import pallas as pl
from jax.experimental.pallas import tpu as pltpu
from jax.experimental.pallas import tpu_sc as plsc

ROWS = 100000
EMB = 128
TC_BLOCK = 10000

_SC_MESH = plsc.VectorSubcoreMesh(core_axis_name="c", subcore_axis_name="s")
_SC_SLICES = _SC_MESH.num_cores * _SC_MESH.num_subcores


def _tc_copy_body(in_ref, out_ref):
    out_ref[...] = in_ref[...]


def _tc_copy(x):
    spec = pl.BlockSpec((TC_BLOCK, EMB), lambda n: (n, 0))
    return pl.pallas_call(
        _tc_copy_body,
        grid=(ROWS // TC_BLOCK,),
        in_specs=[spec],
        out_specs=spec,
        out_shape=jax.ShapeDtypeStruct((ROWS, EMB), jnp.float32),
    )(x)


def _sc_copy(x):
    rows_per = -(-ROWS // _SC_SLICES // 8) * 8

    @pl.kernel(
        out_type=jax.ShapeDtypeStruct((ROWS, EMB), jnp.float32),
        mesh=_SC_MESH,
    )
    def sc_kernel(in_hbm, out_hbm):
        c = jax.lax.axis_index("c")
        s = jax.lax.axis_index("s")
        idx = c * _SC_MESH.num_subcores + s
        start = jnp.minimum(idx * rows_per, ROWS - rows_per)
        start = pl.multiple_of(start, 8)
        pltpu.sync_copy(
            in_hbm.at[pl.ds(start, rows_per), :],
            out_hbm.at[pl.ds(start, rows_per), :],
        )

    return sc_kernel(x)


def kernel(adj, user_emb, item_emb):
    del adj
    return (_tc_copy(user_emb), _sc_copy(item_emb))
